# baseline (device time: 485680 ns/iter reference)
import jax
import jax.numpy as jnp
from jax import lax
from jax.experimental import pallas as pl
from jax.experimental.pallas import tpu as pltpu

C = 4


def kernel(x):
    m, n = x.shape
    P = m // 8
    ch = P // C
    S = m // ch

    def body(x_hbm, out_hbm, ob, in_r,
             ld, oo, xs, xr, ys, yr, zsL, zsR, zfL, zfR):
        my_x = lax.axis_index("x")
        my_y = lax.axis_index("y")
        my_z = lax.axis_index("z")
        mb = my_x * m
        rb = (1 - my_x) * m
        my_po = (my_y * 4 + my_z) * P
        xpart = (1 - my_x, my_y, my_z)
        ymate = (my_x, 1 - my_y, my_z)
        left = (my_x, my_y, my_z - 1)
        right = (my_x, my_y, my_z + 1)

        barrier = pltpu.get_barrier_semaphore()
        for nbr in (xpart, ymate):
            pl.semaphore_signal(
                barrier, inc=1, device_id=nbr,
                device_id_type=pl.DeviceIdType.MESH,
            )

        @pl.when(my_z > 0)
        def _():
            pl.semaphore_signal(
                barrier, inc=1, device_id=left,
                device_id_type=pl.DeviceIdType.MESH,
            )

        @pl.when(my_z < 3)
        def _():
            pl.semaphore_signal(
                barrier, inc=1, device_id=right,
                device_id_type=pl.DeviceIdType.MESH,
            )

        n_nbrs = 2 + jnp.where(my_z > 0, 1, 0) + jnp.where(my_z < 3, 1, 0)
        pl.semaphore_wait(barrier, n_nbrs)

        def slot_off(s):
            return lax.rem(my_po + s * ch, m)

        def load(s):
            cp = pltpu.make_async_copy(
                x_hbm.at[pl.ds(slot_off(s), ch), :],
                in_r.at[s % 2], ld.at[s % 2],
            )
            cp.start()
            return cp

        loads = {s: load(s) for s in range(2)}
        own_outs = []

        def own_slot(s):
            loads[s].wait()
            if s + 2 < S:
                loads[s + 2] = load(s + 2)
            off = slot_off(s)
            ob[pl.ds(off, ch), :] = in_r[s % 2].astype(jnp.bfloat16)
            cp = pltpu.make_async_copy(
                ob.at[pl.ds(off, ch), :],
                out_hbm.at[pl.ds(mb + off, ch), :],
                oo.at[s],
            )
            cp.start()
            own_outs.append(cp)

        descs_x = []
        for c in range(C):
            own_slot(c)
            rd = pltpu.make_async_remote_copy(
                src_ref=ob.at[pl.ds(my_po + c * ch, ch), :],
                dst_ref=out_hbm.at[pl.ds(mb + my_po + c * ch, ch), :],
                send_sem=xs.at[c], recv_sem=xr.at[c],
                device_id=xpart, device_id_type=pl.DeviceIdType.MESH,
            )
            rd.start()
            descs_x.append(rd)

        drains = []
        own_next = C
        for c in range(C):
            for z0 in range(4):
                for y0 in range(2):
                    po0 = (y0 * 4 + z0) * P
                    goff = rb + po0 + c * ch
                    mine = jnp.logical_and(my_y == y0, my_z == z0)
                    mate = jnp.logical_and(my_y != y0, my_z == z0)
                    fromL = z0 < my_z
                    fromR = z0 > my_z
                    condR = jnp.logical_and(z0 <= my_z, my_z < 3)
                    condL = jnp.logical_and(z0 >= my_z, my_z > 0)

                    dy = pltpu.make_async_remote_copy(
                        src_ref=out_hbm.at[pl.ds(goff, ch), :],
                        dst_ref=out_hbm.at[pl.ds(goff, ch), :],
                        send_sem=ys.at[c], recv_sem=yr.at[c],
                        device_id=ymate, device_id_type=pl.DeviceIdType.MESH,
                    )
                    dzR = pltpu.make_async_remote_copy(
                        src_ref=out_hbm.at[pl.ds(goff, ch), :],
                        dst_ref=out_hbm.at[pl.ds(goff, ch), :],
                        send_sem=zsR.at[y0, z0, c], recv_sem=zfL.at[y0, z0, c],
                        device_id=right, device_id_type=pl.DeviceIdType.MESH,
                    )
                    dzL = pltpu.make_async_remote_copy(
                        src_ref=out_hbm.at[pl.ds(goff, ch), :],
                        dst_ref=out_hbm.at[pl.ds(goff, ch), :],
                        send_sem=zsL.at[y0, z0, c], recv_sem=zfR.at[y0, z0, c],
                        device_id=left, device_id_type=pl.DeviceIdType.MESH,
                    )

                    @pl.when(mine)
                    def _(c=c):
                        descs_x[c].wait_recv()

                    @pl.when(mate)
                    def _(dy=dy):
                        dy.wait_recv()

                    @pl.when(fromL)
                    def _(dzR=dzR):
                        dzR.wait_recv()

                    @pl.when(fromR)
                    def _(dzL=dzL):
                        dzL.wait_recv()

                    @pl.when(condR)
                    def _(dzR=dzR):
                        dzR.start()

                    @pl.when(condL)
                    def _(dzL=dzL):
                        dzL.start()

                    @pl.when(mine)
                    def _(dy=dy):
                        dy.start()

                    drains.append((mine, dy))
                    drains.append((condR, dzR))
                    drains.append((condL, dzL))

                    if own_next < S:
                        own_slot(own_next)
                        own_next += 1

        for c in range(C):
            descs_x[c].wait_send()
        for cond, d in drains:
            @pl.when(cond)
            def _(d=d):
                d.wait_send()
        for cp in own_outs:
            cp.wait()

    return pl.pallas_call(
        body,
        out_shape=jax.ShapeDtypeStruct((2 * m, n), jnp.bfloat16),
        in_specs=[pl.BlockSpec(memory_space=pl.ANY)],
        out_specs=pl.BlockSpec(memory_space=pl.ANY),
        scratch_shapes=[
            pltpu.VMEM((m, n), jnp.bfloat16),
            pltpu.VMEM((2, P // C, n), jnp.float32),
            pltpu.SemaphoreType.DMA((2,)),
            pltpu.SemaphoreType.DMA((m // (P // C),)),
            pltpu.SemaphoreType.DMA((C,)),
            pltpu.SemaphoreType.DMA((C,)),
            pltpu.SemaphoreType.DMA((C,)),
            pltpu.SemaphoreType.DMA((C,)),
            pltpu.SemaphoreType.DMA((2, 4, C)),
            pltpu.SemaphoreType.DMA((2, 4, C)),
            pltpu.SemaphoreType.DMA((2, 4, C)),
            pltpu.SemaphoreType.DMA((2, 4, C)),
        ],
        compiler_params=pltpu.CompilerParams(collective_id=0),
    )(x)


# device time: 198371 ns/iter; 2.4483x vs baseline; 2.4483x over previous
import jax
import jax.numpy as jnp
from jax import lax
from jax.experimental import pallas as pl
from jax.experimental.pallas import tpu as pltpu

C = 4


def kernel(x):
    m, n = x.shape
    P = m // 8
    ch = P // C
    S = m // ch

    def body(x_hbm, out_hbm, ob, in_r,
             ld, oo, xs, xr, ys, yr, zsL, zsR, zfL, zfR):
        my_x = lax.axis_index("x")
        my_y = lax.axis_index("y")
        my_z = lax.axis_index("z")
        mb = my_x * m
        rb = (1 - my_x) * m
        my_po = (my_y * 4 + my_z) * P
        xpart = (1 - my_x, my_y, my_z)
        ymate = (my_x, 1 - my_y, my_z)
        left = (my_x, my_y, my_z - 1)
        right = (my_x, my_y, my_z + 1)

        barrier = pltpu.get_barrier_semaphore()
        for nbr in (xpart, ymate):
            pl.semaphore_signal(
                barrier, inc=1, device_id=nbr,
                device_id_type=pl.DeviceIdType.MESH,
            )

        @pl.when(my_z > 0)
        def _():
            pl.semaphore_signal(
                barrier, inc=1, device_id=left,
                device_id_type=pl.DeviceIdType.MESH,
            )

        @pl.when(my_z < 3)
        def _():
            pl.semaphore_signal(
                barrier, inc=1, device_id=right,
                device_id_type=pl.DeviceIdType.MESH,
            )

        n_nbrs = 2 + jnp.where(my_z > 0, 1, 0) + jnp.where(my_z < 3, 1, 0)
        pl.semaphore_wait(barrier, n_nbrs)

        def slot_off(s):
            return lax.rem(my_po + s * ch, m)

        def load(s):
            cp = pltpu.make_async_copy(
                x_hbm.at[pl.ds(slot_off(s), ch), :],
                in_r.at[s % 2], ld.at[s % 2],
            )
            cp.start()
            return cp

        loads = {s: load(s) for s in range(2)}
        own_outs = []

        def own_slot(s):
            loads[s].wait()
            if s + 2 < S:
                loads[s + 2] = load(s + 2)
            off = slot_off(s)
            ob[pl.ds(off, ch), :] = in_r[s % 2].astype(jnp.bfloat16)
            cp = pltpu.make_async_copy(
                ob.at[pl.ds(off, ch), :],
                out_hbm.at[pl.ds(mb + off, ch), :],
                oo.at[s],
            )
            cp.start()
            own_outs.append(cp)

        descs_x = []
        for c in range(C):
            own_slot(c)
            rd = pltpu.make_async_remote_copy(
                src_ref=ob.at[pl.ds(my_po + c * ch, ch), :],
                dst_ref=out_hbm.at[pl.ds(mb + my_po + c * ch, ch), :],
                send_sem=xs.at[c], recv_sem=xr.at[c],
                device_id=xpart, device_id_type=pl.DeviceIdType.MESH,
            )
            rd.start()
            descs_x.append(rd)

        drains = []
        own_next = C

        def make_descs(y0, z0, c):
            po0 = (y0 * 4 + z0) * P
            goff = rb + po0 + c * ch
            dy = pltpu.make_async_remote_copy(
                src_ref=out_hbm.at[pl.ds(goff, ch), :],
                dst_ref=out_hbm.at[pl.ds(goff, ch), :],
                send_sem=ys.at[c], recv_sem=yr.at[c],
                device_id=ymate, device_id_type=pl.DeviceIdType.MESH,
            )
            dzR = pltpu.make_async_remote_copy(
                src_ref=out_hbm.at[pl.ds(goff, ch), :],
                dst_ref=out_hbm.at[pl.ds(goff, ch), :],
                send_sem=zsR.at[y0, z0, c], recv_sem=zfL.at[y0, z0, c],
                device_id=right, device_id_type=pl.DeviceIdType.MESH,
            )
            dzL = pltpu.make_async_remote_copy(
                src_ref=out_hbm.at[pl.ds(goff, ch), :],
                dst_ref=out_hbm.at[pl.ds(goff, ch), :],
                send_sem=zsL.at[y0, z0, c], recv_sem=zfR.at[y0, z0, c],
                device_id=left, device_id_type=pl.DeviceIdType.MESH,
            )
            return dy, dzR, dzL

        for c in range(C):
            for z0 in range(4):
                for y0 in range(2):
                    mine = jnp.logical_and(my_y == y0, my_z == z0)
                    dy, dzR, dzL = make_descs(y0, z0, c)

                    @pl.when(mine)
                    def _(c=c, dy=dy):
                        descs_x[c].wait_recv()
                        dy.start()

                    cR = jnp.logical_and(mine, my_z < 3)
                    cL = jnp.logical_and(mine, my_z > 0)

                    @pl.when(cR)
                    def _(dzR=dzR):
                        dzR.start()

                    @pl.when(cL)
                    def _(dzL=dzL):
                        dzL.start()

                    drains += [(mine, dy), (cR, dzR), (cL, dzL)]

        for c in range(C):
            for z0 in range(4):
                for y0 in range(2):
                    mate = jnp.logical_and(my_y != y0, my_z == z0)
                    dy, dzR, dzL = make_descs(y0, z0, c)

                    @pl.when(mate)
                    def _(dy=dy):
                        dy.wait_recv()

                    cR = jnp.logical_and(mate, my_z < 3)
                    cL = jnp.logical_and(mate, my_z > 0)

                    @pl.when(cR)
                    def _(dzR=dzR):
                        dzR.start()

                    @pl.when(cL)
                    def _(dzL=dzL):
                        dzL.start()

                    drains += [(cR, dzR), (cL, dzL)]

                    if y0 == 0 and own_next < S:
                        own_slot(own_next)
                        own_next += 1

        for c in range(C):
            for z0 in range(4):
                for y0 in range(2):
                    fromL = z0 < my_z
                    fromR = z0 > my_z
                    dy, dzR, dzL = make_descs(y0, z0, c)

                    @pl.when(fromL)
                    def _(dzR=dzR):
                        dzR.wait_recv()

                    @pl.when(fromR)
                    def _(dzL=dzL):
                        dzL.wait_recv()

                    cR = jnp.logical_and(fromL, my_z < 3)
                    cL = jnp.logical_and(fromR, my_z > 0)

                    @pl.when(cR)
                    def _(dzR=dzR):
                        dzR.start()

                    @pl.when(cL)
                    def _(dzL=dzL):
                        dzL.start()

                    drains += [(cR, dzR), (cL, dzL)]

                    if y0 == 0 and own_next < S:
                        own_slot(own_next)
                        own_next += 1

        for c in range(C):
            descs_x[c].wait_send()
        for cond, d in drains:
            @pl.when(cond)
            def _(d=d):
                d.wait_send()
        for cp in own_outs:
            cp.wait()

    return pl.pallas_call(
        body,
        out_shape=jax.ShapeDtypeStruct((2 * m, n), jnp.bfloat16),
        in_specs=[pl.BlockSpec(memory_space=pl.ANY)],
        out_specs=pl.BlockSpec(memory_space=pl.ANY),
        scratch_shapes=[
            pltpu.VMEM((m, n), jnp.bfloat16),
            pltpu.VMEM((2, P // C, n), jnp.float32),
            pltpu.SemaphoreType.DMA((2,)),
            pltpu.SemaphoreType.DMA((m // (P // C),)),
            pltpu.SemaphoreType.DMA((C,)),
            pltpu.SemaphoreType.DMA((C,)),
            pltpu.SemaphoreType.DMA((C,)),
            pltpu.SemaphoreType.DMA((C,)),
            pltpu.SemaphoreType.DMA((2, 4, C)),
            pltpu.SemaphoreType.DMA((2, 4, C)),
            pltpu.SemaphoreType.DMA((2, 4, C)),
            pltpu.SemaphoreType.DMA((2, 4, C)),
        ],
        compiler_params=pltpu.CompilerParams(collective_id=0),
    )(x)
